# baseline (speedup 1.0000x reference)
import functools
import math

import jax
import jax.numpy as jnp
import numpy as np
from jax import lax
from jax.experimental import pallas as pl
from jax.experimental.pallas import tpu as pltpu
from jax.experimental.pallas import tpu_sc as plsc

D_MODEL = 128
CONTEXT = 2048
B, S = 32, 2048

NC, NS = 2, 16
NW = NC * NS
BAND = S // NW
GATHER_ROWS = 128
CHUNK_ROWS = 512
GATHERS_PER_CHUNK = CHUNK_ROWS // GATHER_ROWS
N_GATHERS = (B * BAND) // GATHER_ROWS
N_CHUNKS = (B * BAND) // CHUNK_ROWS
BATCHES_PER_CHUNK = CHUNK_ROWS // BAND
BATCHES_PER_GATHER = GATHER_ROWS // BAND
NLANE = 16


def _make_pe2():
    position = np.arange(CONTEXT, dtype=np.float32)[:, None]
    div_term = np.exp(
        np.arange(0, D_MODEL, 2, dtype=np.float32) * (-math.log(10000.0) / D_MODEL)
    )
    pe = np.zeros((CONTEXT, D_MODEL), dtype=np.float32)
    pe[:, 0::2] = np.sin(position * div_term)
    pe[:, 1::2] = np.cos(position * div_term)
    return 2.0 * pe


_PE2 = _make_pe2()

_mesh = plsc.VectorSubcoreMesh(core_axis_name="c", subcore_axis_name="s")


@functools.partial(
    pl.kernel,
    mesh=_mesh,
    out_type=jax.ShapeDtypeStruct((B * S, D_MODEL), jnp.float32),
    scratch_types=[
        pltpu.VMEM((N_GATHERS, GATHER_ROWS), jnp.int32),
        pltpu.VMEM((BAND, D_MODEL), jnp.float32),
        pltpu.VMEM((CHUNK_ROWS, D_MODEL), jnp.float32),
        pltpu.SemaphoreType.DMA,
    ],
)
def _embed(table_hbm, x_hbm, pe2_hbm, out_hbm, idx_v, pe_v, rows_v, sem):
    wid = lax.axis_index("s") * NC + lax.axis_index("c")
    p0 = wid * BAND

    for b in range(B):
        pltpu.sync_copy(
            x_hbm.at[pl.ds(b * S + p0, BAND)],
            idx_v.at[b // BATCHES_PER_GATHER, pl.ds((b % BATCHES_PER_GATHER) * BAND, BAND)],
        )
    pltpu.sync_copy(pe2_hbm.at[pl.ds(p0, BAND), :], pe_v)

    for c in range(N_CHUNKS):
        descs = [
            pltpu.async_copy(
                table_hbm.at[idx_v.at[c * GATHERS_PER_CHUNK + q]],
                rows_v.at[pl.ds(q * GATHER_ROWS, GATHER_ROWS)],
                sem,
            )
            for q in range(GATHERS_PER_CHUNK)
        ]
        for d in descs:
            d.wait()

        def body(r, _):
            rp = lax.rem(r, BAND)
            for j in range(D_MODEL // NLANE):
                sl = pl.ds(j * NLANE, NLANE)
                e = rows_v[r, sl]
                p = pe_v[rp, sl]
                rows_v[r, sl] = e + e + p
            return 0

        lax.fori_loop(0, CHUNK_ROWS, body, 0)

        for k in range(BATCHES_PER_CHUNK):
            b = c * BATCHES_PER_CHUNK + k
            pltpu.sync_copy(
                rows_v.at[pl.ds(k * BAND, BAND)],
                out_hbm.at[pl.ds(b * S + p0, BAND)],
            )


def kernel(x, table):
    x_flat = x.reshape(-1).astype(jnp.int32)
    pe2 = jnp.asarray(_PE2)
    out = _embed(table, x_flat, pe2)
    return out.reshape(B, S, D_MODEL)


# speedup vs baseline: 2.7976x; 2.7976x over previous
import functools
import math

import jax
import jax.numpy as jnp
import numpy as np
from jax import lax
from jax.experimental import pallas as pl
from jax.experimental.pallas import tpu as pltpu
from jax.experimental.pallas import tpu_sc as plsc

D_MODEL = 128
CONTEXT = 2048
B, S = 32, 2048

NC, NS = 2, 16
NW = NC * NS
BAND = S // NW
GATHER_ROWS = 128
CHUNK_ROWS = 256
GATHERS_PER_CHUNK = CHUNK_ROWS // GATHER_ROWS
N_GATHERS = (B * BAND) // GATHER_ROWS
N_CHUNKS = (B * BAND) // CHUNK_ROWS
BATCHES_PER_CHUNK = CHUNK_ROWS // BAND
NBUF = 2
NLANE = 16
NCOL = D_MODEL // NLANE


def _make_pe2():
    position = np.arange(CONTEXT, dtype=np.float32)[:, None]
    div_term = np.exp(
        np.arange(0, D_MODEL, 2, dtype=np.float32) * (-math.log(10000.0) / D_MODEL)
    )
    pe = np.zeros((CONTEXT, D_MODEL), dtype=np.float32)
    pe[:, 0::2] = np.sin(position * div_term)
    pe[:, 1::2] = np.cos(position * div_term)
    return 2.0 * pe


_PE2 = _make_pe2()

_mesh = plsc.VectorSubcoreMesh(core_axis_name="c", subcore_axis_name="s")


@functools.partial(
    pl.kernel,
    mesh=_mesh,
    out_type=jax.ShapeDtypeStruct((B * S, D_MODEL), jnp.float32),
    scratch_types=[
        pltpu.VMEM((N_GATHERS, GATHER_ROWS), jnp.int32),
        pltpu.VMEM((BAND, D_MODEL), jnp.float32),
        pltpu.VMEM((NBUF, CHUNK_ROWS, D_MODEL), jnp.float32),
        pltpu.SemaphoreType.DMA,
        pltpu.SemaphoreType.DMA,
        pltpu.SemaphoreType.DMA,
        pltpu.SemaphoreType.DMA,
    ],
)
def _embed(table_hbm, x_hbm, pe2_hbm, out_hbm, idx_v, pe_v, rows_v, sem_idx,
           sem_pe, sem_g, sem_w):
    wid = lax.axis_index("s") * NC + lax.axis_index("c")
    p0 = wid * BAND

    idx_descs = [
        pltpu.async_copy(
            x_hbm.at[pl.ds(b * S + p0, BAND)],
            idx_v.at[b // 2, pl.ds((b % 2) * BAND, BAND)],
            sem_idx,
        )
        for b in range(B)
    ]
    pe_desc = pltpu.async_copy(pe2_hbm.at[pl.ds(p0, BAND), :], pe_v, sem_pe)
    for d in idx_descs:
        d.wait()

    def fire_gather(c):
        return [
            pltpu.async_copy(
                table_hbm.at[idx_v.at[c * GATHERS_PER_CHUNK + q]],
                rows_v.at[c % NBUF, pl.ds(q * GATHER_ROWS, GATHER_ROWS)],
                sem_g,
            )
            for q in range(GATHERS_PER_CHUNK)
        ]

    def fire_writes(c):
        return [
            pltpu.async_copy(
                rows_v.at[c % NBUF, pl.ds(kb * BAND, BAND)],
                out_hbm.at[pl.ds((c * BATCHES_PER_CHUNK + kb) * S + p0, BAND)],
                sem_w,
            )
            for kb in range(BATCHES_PER_CHUNK)
        ]

    def compute(c):
        buf = c % NBUF

        @plsc.parallel_loop(0, BAND)
        def _(i):
            for j in range(NCOL):
                sl = pl.ds(j * NLANE, NLANE)
                p = pe_v[i, sl]
                for kb in range(BATCHES_PER_CHUNK):
                    r = kb * BAND + i
                    e = rows_v[buf, r, sl]
                    rows_v[buf, r, sl] = e + e + p

    g_descs = {0: fire_gather(0)}
    w_descs = {}
    pe_desc.wait()
    for c in range(N_CHUNKS):
        if c + 1 < N_CHUNKS:
            if c - 1 >= 0:
                for d in w_descs.pop(c - 1):
                    d.wait()
            g_descs[c + 1] = fire_gather(c + 1)
        for d in g_descs.pop(c):
            d.wait()
        compute(c)
        w_descs[c] = fire_writes(c)
    for ds in w_descs.values():
        for d in ds:
            d.wait()


def kernel(x, table):
    x_flat = x.reshape(-1).astype(jnp.int32)
    pe2 = jnp.asarray(_PE2)
    out = _embed(table, x_flat, pe2)
    return out.reshape(B, S, D_MODEL)


# speedup vs baseline: 2.8624x; 1.0231x over previous
import functools
import math

import jax
import jax.numpy as jnp
import numpy as np
from jax import lax
from jax.experimental import pallas as pl
from jax.experimental.pallas import tpu as pltpu
from jax.experimental.pallas import tpu_sc as plsc

D_MODEL = 128
CONTEXT = 2048
B, S = 32, 2048

NC, NS = 2, 16
NW = NC * NS
GB = 16
PW = 128
N_BGROUP = B // GB
N_PBAND = S // PW
GATHER_ROWS = 128
CHUNK_ROWS = 256
GATHERS_PER_CHUNK = CHUNK_ROWS // GATHER_ROWS
N_GATHERS = (GB * PW) // GATHER_ROWS
N_CHUNKS = (GB * PW) // CHUNK_ROWS
BATCHES_PER_CHUNK = CHUNK_ROWS // PW
NBUF = 3
NLANE = 16
NCOL = D_MODEL // NLANE


def _make_pe2():
    position = np.arange(CONTEXT, dtype=np.float32)[:, None]
    div_term = np.exp(
        np.arange(0, D_MODEL, 2, dtype=np.float32) * (-math.log(10000.0) / D_MODEL)
    )
    pe = np.zeros((CONTEXT, D_MODEL), dtype=np.float32)
    pe[:, 0::2] = np.sin(position * div_term)
    pe[:, 1::2] = np.cos(position * div_term)
    return 2.0 * pe


_PE2 = _make_pe2()

_mesh = plsc.VectorSubcoreMesh(core_axis_name="c", subcore_axis_name="s")


@functools.partial(
    pl.kernel,
    mesh=_mesh,
    out_type=jax.ShapeDtypeStruct((B * S, D_MODEL), jnp.float32),
    scratch_types=[
        pltpu.VMEM((N_GATHERS, GATHER_ROWS), jnp.int32),
        pltpu.VMEM((PW, D_MODEL), jnp.float32),
        pltpu.VMEM((NBUF, CHUNK_ROWS, D_MODEL), jnp.float32),
        pltpu.SemaphoreType.DMA,
        pltpu.SemaphoreType.DMA,
        pltpu.SemaphoreType.DMA,
        pltpu.SemaphoreType.DMA,
    ],
)
def _embed(table_hbm, x_hbm, pe2_hbm, out_hbm, idx_v, pe_v, rows_v, sem_idx,
           sem_pe, sem_g, sem_w):
    wid = lax.axis_index("s") * NC + lax.axis_index("c")
    bg = wid // N_PBAND
    pb = wid % N_PBAND
    b0 = bg * GB
    p0 = pb * PW

    idx_descs = [
        pltpu.async_copy(
            x_hbm.at[pl.ds((b0 + r) * S + p0, PW)],
            idx_v.at[r],
            sem_idx,
        )
        for r in range(N_GATHERS)
    ]
    pe_desc = pltpu.async_copy(pe2_hbm.at[pl.ds(p0, PW), :], pe_v, sem_pe)
    for d in idx_descs:
        d.wait()

    def fire_gather(c):
        return [
            pltpu.async_copy(
                table_hbm.at[idx_v.at[c * GATHERS_PER_CHUNK + q]],
                rows_v.at[c % NBUF, pl.ds(q * GATHER_ROWS, GATHER_ROWS)],
                sem_g,
            )
            for q in range(GATHERS_PER_CHUNK)
        ]

    def fire_writes(c):
        return [
            pltpu.async_copy(
                rows_v.at[c % NBUF, pl.ds(kb * PW, PW)],
                out_hbm.at[
                    pl.ds((b0 + c * BATCHES_PER_CHUNK + kb) * S + p0, PW)
                ],
                sem_w,
            )
            for kb in range(BATCHES_PER_CHUNK)
        ]

    def compute(c):
        buf = c % NBUF

        @plsc.parallel_loop(0, PW, unroll=2)
        def _(i):
            for j in range(NCOL):
                sl = pl.ds(j * NLANE, NLANE)
                p = pe_v[i, sl]
                for kb in range(BATCHES_PER_CHUNK):
                    r = kb * PW + i
                    e = rows_v[buf, r, sl]
                    rows_v[buf, r, sl] = e + e + p

    g_descs = {0: fire_gather(0), 1: fire_gather(1)}
    w_descs = {}
    pe_desc.wait()
    for c in range(N_CHUNKS):
        for d in g_descs.pop(c):
            d.wait()
        compute(c)
        w_descs[c] = fire_writes(c)
        if c + 2 < N_CHUNKS:
            if c - 1 >= 0:
                for d in w_descs.pop(c - 1):
                    d.wait()
            g_descs[c + 2] = fire_gather(c + 2)
    for ds in w_descs.values():
        for d in ds:
            d.wait()


def kernel(x, table):
    x_flat = x.reshape(-1).astype(jnp.int32)
    pe2 = jnp.asarray(_PE2)
    out = _embed(table, x_flat, pe2)
    return out.reshape(B, S, D_MODEL)


# speedup vs baseline: 2.8889x; 1.0093x over previous
import functools
import math

import jax
import jax.numpy as jnp
import numpy as np
from jax import lax
from jax.experimental import pallas as pl
from jax.experimental.pallas import tpu as pltpu
from jax.experimental.pallas import tpu_sc as plsc

D_MODEL = 128
CONTEXT = 2048
B, S = 32, 2048

NC, NS = 2, 16
NW = NC * NS
GB = 16
PW = 128
N_BGROUP = B // GB
N_PBAND = S // PW
GATHER_ROWS = 128
CHUNK_ROWS = 256
GATHERS_PER_CHUNK = CHUNK_ROWS // GATHER_ROWS
N_GATHERS = (GB * PW) // GATHER_ROWS
N_CHUNKS = (GB * PW) // CHUNK_ROWS
BATCHES_PER_CHUNK = CHUNK_ROWS // PW
NBUF = 3
NLANE = 16
NCOL = D_MODEL // NLANE


def _make_pe2():
    position = np.arange(CONTEXT, dtype=np.float32)[:, None]
    div_term = np.exp(
        np.arange(0, D_MODEL, 2, dtype=np.float32) * (-math.log(10000.0) / D_MODEL)
    )
    pe = np.zeros((CONTEXT, D_MODEL), dtype=np.float32)
    pe[:, 0::2] = np.sin(position * div_term)
    pe[:, 1::2] = np.cos(position * div_term)
    return 2.0 * pe


_PE2 = _make_pe2()

_mesh = plsc.VectorSubcoreMesh(core_axis_name="c", subcore_axis_name="s")


@functools.partial(
    pl.kernel,
    mesh=_mesh,
    out_type=jax.ShapeDtypeStruct((B, S, D_MODEL), jnp.float32),
    scratch_types=[
        pltpu.VMEM((N_GATHERS, GATHER_ROWS), jnp.int32),
        pltpu.VMEM((PW, D_MODEL), jnp.float32),
        pltpu.VMEM((NBUF, CHUNK_ROWS, D_MODEL), jnp.float32),
        pltpu.SemaphoreType.DMA,
        pltpu.SemaphoreType.DMA,
        pltpu.SemaphoreType.DMA,
        pltpu.SemaphoreType.DMA,
    ],
)
def _embed(table_hbm, x_hbm, pe2_hbm, out_hbm, idx_v, pe_v, rows_v, sem_idx,
           sem_pe, sem_g, sem_w):
    wid = lax.axis_index("s") * NC + lax.axis_index("c")
    bg = wid // N_PBAND
    pb = wid % N_PBAND
    b0 = bg * GB
    p0 = pb * PW

    idx_descs = [
        pltpu.async_copy(
            x_hbm.at[b0 + r, pl.ds(p0, PW)],
            idx_v.at[r],
            sem_idx,
        )
        for r in range(N_GATHERS)
    ]
    pe_desc = pltpu.async_copy(pe2_hbm.at[pl.ds(p0, PW), :], pe_v, sem_pe)
    for d in idx_descs:
        d.wait()

    def fire_gather(c):
        return [
            pltpu.async_copy(
                table_hbm.at[idx_v.at[c * GATHERS_PER_CHUNK + q]],
                rows_v.at[c % NBUF, pl.ds(q * GATHER_ROWS, GATHER_ROWS)],
                sem_g,
            )
            for q in range(GATHERS_PER_CHUNK)
        ]

    def fire_writes(c):
        return [
            pltpu.async_copy(
                rows_v.at[c % NBUF, pl.ds(kb * PW, PW)],
                out_hbm.at[b0 + c * BATCHES_PER_CHUNK + kb, pl.ds(p0, PW), :],
                sem_w,
            )
            for kb in range(BATCHES_PER_CHUNK)
        ]

    def compute(c):
        buf = c % NBUF

        @plsc.parallel_loop(0, PW, unroll=2)
        def _(i):
            for j in range(NCOL):
                sl = pl.ds(j * NLANE, NLANE)
                p = pe_v[i, sl]
                for kb in range(BATCHES_PER_CHUNK):
                    r = kb * PW + i
                    e = rows_v[buf, r, sl]
                    rows_v[buf, r, sl] = e + e + p

    g_descs = {0: fire_gather(0), 1: fire_gather(1)}
    w_descs = {}
    pe_desc.wait()
    for c in range(N_CHUNKS):
        for d in g_descs.pop(c):
            d.wait()
        compute(c)
        w_descs[c] = fire_writes(c)
        if c + 2 < N_CHUNKS:
            if c - 1 >= 0:
                for d in w_descs.pop(c - 1):
                    d.wait()
            g_descs[c + 2] = fire_gather(c + 2)
    for ds in w_descs.values():
        for d in ds:
            d.wait()


def kernel(x, table):
    pe2 = jnp.asarray(_PE2)
    return _embed(table, x.astype(jnp.int32), pe2)


# speedup vs baseline: 2.9248x; 1.0124x over previous
import functools
import math

import jax
import jax.numpy as jnp
import numpy as np
from jax import lax
from jax.experimental import pallas as pl
from jax.experimental.pallas import tpu as pltpu
from jax.experimental.pallas import tpu_sc as plsc

D_MODEL = 128
CONTEXT = 2048
B, S = 32, 2048

NC, NS = 2, 16
NW = NC * NS
GB = 16
PW = 128
N_BGROUP = B // GB
N_PBAND = S // PW
GATHER_ROWS = 128
CHUNK_ROWS = 128
N_GATHERS = (GB * PW) // GATHER_ROWS
N_CHUNKS = (GB * PW) // CHUNK_ROWS
NBUF = 6
LOOKAHEAD = NBUF - 2
NLANE = 16
NCOL = D_MODEL // NLANE


def _make_pe2():
    position = np.arange(CONTEXT, dtype=np.float32)[:, None]
    div_term = np.exp(
        np.arange(0, D_MODEL, 2, dtype=np.float32) * (-math.log(10000.0) / D_MODEL)
    )
    pe = np.zeros((CONTEXT, D_MODEL), dtype=np.float32)
    pe[:, 0::2] = np.sin(position * div_term)
    pe[:, 1::2] = np.cos(position * div_term)
    return 2.0 * pe


_PE2 = _make_pe2()

_mesh = plsc.VectorSubcoreMesh(core_axis_name="c", subcore_axis_name="s")


@functools.partial(
    pl.kernel,
    mesh=_mesh,
    out_type=jax.ShapeDtypeStruct((B, S, D_MODEL), jnp.float32),
    scratch_types=[
        pltpu.VMEM((N_GATHERS, GATHER_ROWS), jnp.int32),
        pltpu.VMEM((PW, D_MODEL), jnp.float32),
        pltpu.VMEM((NBUF, CHUNK_ROWS, D_MODEL), jnp.float32),
        pltpu.SemaphoreType.DMA,
        pltpu.SemaphoreType.DMA,
        pltpu.SemaphoreType.DMA,
        pltpu.SemaphoreType.DMA,
    ],
)
def _embed(table_hbm, x_hbm, pe2_hbm, out_hbm, idx_v, pe_v, rows_v, sem_idx,
           sem_pe, sem_g, sem_w):
    wid = lax.axis_index("s") * NC + lax.axis_index("c")
    bg = wid // N_PBAND
    pb = wid % N_PBAND
    b0 = bg * GB
    p0 = pb * PW

    idx_descs = [
        pltpu.async_copy(
            x_hbm.at[b0 + r, pl.ds(p0, PW)],
            idx_v.at[r],
            sem_idx,
        )
        for r in range(N_GATHERS)
    ]
    pe_desc = pltpu.async_copy(pe2_hbm.at[pl.ds(p0, PW), :], pe_v, sem_pe)
    for d in idx_descs:
        d.wait()

    def fire_gather(c):
        return pltpu.async_copy(
            table_hbm.at[idx_v.at[c]],
            rows_v.at[c % NBUF],
            sem_g,
        )

    def fire_write(c):
        return pltpu.async_copy(
            rows_v.at[c % NBUF],
            out_hbm.at[b0 + c, pl.ds(p0, PW), :],
            sem_w,
        )

    def compute(c):
        buf = c % NBUF

        @plsc.parallel_loop(0, PW, unroll=2)
        def _(i):
            for j in range(NCOL):
                sl = pl.ds(j * NLANE, NLANE)
                p = pe_v[i, sl]
                e = rows_v[buf, i, sl]
                rows_v[buf, i, sl] = e + e + p

    g_descs = {c: fire_gather(c) for c in range(LOOKAHEAD)}
    w_descs = {}
    pe_desc.wait()
    for c in range(N_CHUNKS):
        g_descs.pop(c).wait()
        compute(c)
        w_descs[c] = fire_write(c)
        if c + LOOKAHEAD < N_CHUNKS:
            prev = c + LOOKAHEAD - NBUF
            if prev >= 0:
                w_descs.pop(prev).wait()
            g_descs[c + LOOKAHEAD] = fire_gather(c + LOOKAHEAD)
    for d in w_descs.values():
        d.wait()


def kernel(x, table):
    pe2 = jnp.asarray(_PE2)
    return _embed(table, x.astype(jnp.int32), pe2)
